# Initial kernel scaffold; baseline (speedup 1.0000x reference)
import jax
import jax.numpy as jnp
from jax.experimental import pallas as pl


def kernel(x, edge_index, W_gat, att_src, att_dst, b_gat, W_ih, W_hh, b_ih, b_hh, W_lin, b_lin):
    raise NotImplementedError("write your pallas kernel here")


# speedup vs baseline: 13.0251x; 13.0251x over previous
import functools

import jax
import jax.numpy as jnp
from jax import lax
from jax.experimental import pallas as pl
from jax.experimental.pallas import tpu as pltpu
from jax.experimental.pallas import tpu_sc as plsc

N = 10000
E = 320000
F_IN = 128
HO = 64
HG = 16
NC = 2
NS = 16
NW = NC * NS
CH = 128
E_PAD = ((E + NW * CH - 1) // (NW * CH)) * (NW * CH)
EPW = E_PAD // NW
NCHUNK = EPW // CH
NP = 10240
RPT = NP // NS


def _pre_body(x_ref, wg_ref, a1_ref, a2_ref, xw_ref, t1_ref, t2_ref, c_ref):
    xw = jnp.dot(x_ref[...], wg_ref[...], preferred_element_type=jnp.float32)
    xw_ref[...] = xw
    t1 = jnp.dot(xw, a1_ref[...], preferred_element_type=jnp.float32)
    t2 = jnp.dot(xw, a2_ref[...], preferred_element_type=jnp.float32)
    t1_ref[...] = t1
    t2_ref[...] = t2
    m = (jnp.max(t1, axis=0, keepdims=True)
         + jnp.max(t2, axis=0, keepdims=True))
    c_ref[...] = jnp.maximum(m, 0.2 * m)


def _pre(x, wg, a1, a2):
    return pl.pallas_call(
        _pre_body,
        out_shape=[
            jax.ShapeDtypeStruct((N, HO), jnp.float32),
            jax.ShapeDtypeStruct((N, 16), jnp.float32),
            jax.ShapeDtypeStruct((N, 16), jnp.float32),
            jax.ShapeDtypeStruct((1, 16), jnp.float32),
        ],
    )(x, wg, a1, a2)


def _edge_sc(src, dst, t1, t2, xw, cvec):
    mesh = plsc.VectorSubcoreMesh(core_axis_name="c", subcore_axis_name="s")

    @functools.partial(
        pl.kernel,
        out_type=[
            jax.ShapeDtypeStruct((NC, N, HO), jnp.float32),
            jax.ShapeDtypeStruct((NC, N, 16), jnp.float32),
        ],
        mesh=mesh,
        compiler_params=pltpu.CompilerParams(use_tc_tiling_on_sc=False,
                                             needs_layout_passes=False),
        scratch_types=[
            pltpu.VMEM((CH,), jnp.int32),
            pltpu.VMEM((CH,), jnp.int32),
            pltpu.VMEM((CH, 16), jnp.float32),
            pltpu.VMEM((CH, 16), jnp.float32),
            pltpu.VMEM((CH, HO), jnp.float32),
            pltpu.VMEM((CH, 16), jnp.float32),
            pltpu.VMEM((CH, HO), jnp.float32),
            pltpu.VMEM((16,), jnp.float32),
            pltpu.VMEM_SHARED((NP, HO), jnp.float32),
            pltpu.VMEM_SHARED((NP, 16), jnp.float32),
            pltpu.SemaphoreType.DMA,
        ],
    )
    def k(src_hbm, dst_hbm, t1_hbm, t2_hbm, xw_hbm, c_hbm,
          out_hbm, esum_hbm,
          sidx, didx, t1r, t2r, xwr, exb, msg, cv, out_sp, esum_sp, sem):
        cid = lax.axis_index("c")
        sid = lax.axis_index("s")
        wid = sid * NC + cid

        zero16 = jnp.zeros((16,), jnp.float32)

        def zrow(i, _):
            for j in range(HO // 16):
                msg[i, pl.ds(j * 16, 16)] = zero16
            exb[i, :] = zero16
            return 0

        lax.fori_loop(0, CH, zrow, 0)
        r0 = sid * RPT
        for b in range(RPT // CH):
            pltpu.sync_copy(msg, out_sp.at[pl.ds(r0 + b * CH, CH)])
            pltpu.sync_copy(exb, esum_sp.at[pl.ds(r0 + b * CH, CH)])
        pltpu.sync_copy(c_hbm, cv)
        plsc.subcore_barrier()

        cval = cv[...]
        io = lax.iota(jnp.int32, 16)
        col0 = (io >= 8).astype(jnp.int32)

        def chunk(g, _):
            base = wid * EPW + g * CH
            pltpu.sync_copy(src_hbm.at[pl.ds(base, CH)], sidx)
            pltpu.sync_copy(dst_hbm.at[pl.ds(base, CH)], didx)
            c1 = pltpu.async_copy(t1_hbm.at[sidx], t1r, sem)
            c2 = pltpu.async_copy(t2_hbm.at[didx], t2r, sem)
            c3 = pltpu.async_copy(xw_hbm.at[sidx], xwr, sem)
            c1.wait()
            c2.wait()

            def p1(i, _):
                a = t1r[i, :] + t2r[i, :]
                al = jnp.maximum(a, 0.2 * a)
                exb[i, :] = jnp.exp(al - cval)
                return 0

            lax.fori_loop(0, CH, p1, 0)
            pltpu.sync_copy(exb, esum_sp.at[didx], add=True)
            c3.wait()

            def p2(i, _):
                row = jnp.full((16,), i, jnp.int32)
                for j in range(HO // 16):
                    eb = plsc.load_gather(exb, [row, col0 + 2 * j])
                    msg[i, pl.ds(j * 16, 16)] = xwr[i, pl.ds(j * 16, 16)] * eb
                return 0

            lax.fori_loop(0, CH, p2, 0)
            pltpu.sync_copy(msg, out_sp.at[didx], add=True)
            return 0

        lax.fori_loop(0, NCHUNK, chunk, 0)
        plsc.subcore_barrier()

        nfull = N // RPT
        nlast = N - nfull * RPT

        @pl.when(sid < nfull)
        def _():
            pltpu.sync_copy(out_sp.at[pl.ds(r0, RPT)],
                            out_hbm.at[cid, pl.ds(r0, RPT)])
            pltpu.sync_copy(esum_sp.at[pl.ds(r0, RPT)],
                            esum_hbm.at[cid, pl.ds(r0, RPT)])

        @pl.when(sid == nfull)
        def _():
            pltpu.sync_copy(out_sp.at[pl.ds(nfull * RPT, nlast)],
                            out_hbm.at[cid, pl.ds(nfull * RPT, nlast)])
            pltpu.sync_copy(esum_sp.at[pl.ds(nfull * RPT, nlast)],
                            esum_hbm.at[cid, pl.ds(nfull * RPT, nlast)])

    return k(src, dst, t1, t2, xw, cvec)


def _post_body(outp_ref, esump_ref, xw_ref, t1_ref, t2_ref, c_ref, r8_ref,
               wih_ref, badd_ref, whh_ref, bn_ref, bgat_ref, wlin_ref,
               blin_ref, o_ref, h_ref, gi_scr):
    s = t1_ref[...] + t2_ref[...]
    exs = jnp.exp(jnp.maximum(s, 0.2 * s) - c_ref[...])
    esum = esump_ref[0] + esump_ref[1] + exs
    r8 = r8_ref[...]
    exb = jnp.dot(exs, r8, preferred_element_type=jnp.float32)
    esb = jnp.dot(esum, r8, preferred_element_type=jnp.float32)
    xw = xw_ref[...]
    out = outp_ref[0] + outp_ref[1] + xw * exb
    g = jnp.maximum(out / (esb + 1e-16) + bgat_ref[...], 0.0)
    gi_scr[...] = jnp.dot(g, wih_ref[...],
                          preferred_element_type=jnp.float32) + badd_ref[...]
    whh = whh_ref[...]
    bn = bn_ref[...]

    def step(t, h):
        git = gi_scr[pl.ds(t, 1), :]
        gh = jnp.dot(h, whh, preferred_element_type=jnp.float32)
        sa = git + gh
        r = 1.0 / (1.0 + jnp.exp(-sa[:, 0:16]))
        z = 1.0 / (1.0 + jnp.exp(-sa[:, 16:32]))
        n = jnp.tanh(git[:, 32:48] + r * (gh[:, 32:48] + bn))
        return (1.0 - z) * n + z * h

    h = lax.fori_loop(0, N, step, jnp.zeros((1, HG), jnp.float32))
    h_ref[...] = h
    o_ref[...] = jnp.dot(h, wlin_ref[...],
                         preferred_element_type=jnp.float32) + blin_ref[...]


def _post(outp, esump, xw, t1, t2, cvec, r8, wih, badd, whh, bn, bgat,
          wlin, blin):
    return pl.pallas_call(
        _post_body,
        out_shape=[
            jax.ShapeDtypeStruct((1, 1), jnp.float32),
            jax.ShapeDtypeStruct((1, HG), jnp.float32),
        ],
        scratch_shapes=[pltpu.VMEM((N, 48), jnp.float32)],
    )(outp, esump, xw, t1, t2, cvec, r8, wih, badd, whh, bn, bgat, wlin, blin)


def kernel(x, edge_index, W_gat, att_src, att_dst, b_gat,
           W_ih, W_hh, b_ih, b_hh, W_lin, b_lin):
    f32 = jnp.float32
    oh = (jnp.arange(HO)[:, None] // 8 == jnp.arange(16)[None, :]).astype(f32)
    a1 = oh * att_src.reshape(HO, 1)
    a2 = oh * att_dst.reshape(HO, 1)
    r8 = oh.T

    xw, t1, t2, cvec = _pre(x, W_gat, a1, a2)

    pad = E_PAD - E
    src = jnp.concatenate([edge_index[0].astype(jnp.int32),
                           jnp.zeros((pad,), jnp.int32)])
    dst = jnp.concatenate([edge_index[1].astype(jnp.int32),
                           jnp.full((pad,), N, jnp.int32)])
    outp, esump = _edge_sc(src, dst, t1, t2, xw, cvec.reshape(16))

    wih = W_ih.T
    badd = (b_ih + jnp.concatenate([b_hh[0:32], jnp.zeros((16,), f32)])
            ).reshape(1, 48)
    whh = W_hh.T
    bn = b_hh[32:48].reshape(1, HG)
    o, h = _post(outp, esump, xw, t1, t2, cvec, r8, wih, badd, whh, bn,
                 b_gat.reshape(1, HO), W_lin.T, b_lin.reshape(1, 1))
    return (o, h.reshape(1, 1, HG))


# speedup vs baseline: 35.7590x; 2.7454x over previous
import functools

import jax
import jax.numpy as jnp
from jax import lax
from jax.experimental import pallas as pl
from jax.experimental.pallas import tpu as pltpu
from jax.experimental.pallas import tpu_sc as plsc

N = 10000
E = 320000
F_IN = 128
HO = 64
HG = 16
NC = 2
NS = 16
NW = NC * NS
CH = 128
E_PAD = ((E + NW * CH - 1) // (NW * CH)) * (NW * CH)
EPW = E_PAD // NW
NCHUNK = EPW // CH
NP = 10240
RPT = NP // NS


def _pre_body(x_ref, wg_ref, a1_ref, a2_ref, xw_ref, t1_ref, t2_ref, c_ref):
    xw = jnp.dot(x_ref[...], wg_ref[...], preferred_element_type=jnp.float32)
    xw_ref[...] = xw
    t1 = jnp.dot(xw, a1_ref[...], preferred_element_type=jnp.float32)
    t2 = jnp.dot(xw, a2_ref[...], preferred_element_type=jnp.float32)
    t1_ref[...] = t1
    t2_ref[...] = t2
    m = (jnp.max(t1, axis=0, keepdims=True)
         + jnp.max(t2, axis=0, keepdims=True))
    c_ref[...] = jnp.maximum(m, 0.2 * m)


def _pre(x, wg, a1, a2):
    return pl.pallas_call(
        _pre_body,
        out_shape=[
            jax.ShapeDtypeStruct((N, HO), jnp.float32),
            jax.ShapeDtypeStruct((N, 16), jnp.float32),
            jax.ShapeDtypeStruct((N, 16), jnp.float32),
            jax.ShapeDtypeStruct((1, 16), jnp.float32),
        ],
    )(x, wg, a1, a2)


def _edge_sc(src, dst, t1, t2, xw, cvec):
    mesh = plsc.VectorSubcoreMesh(core_axis_name="c", subcore_axis_name="s")

    @functools.partial(
        pl.kernel,
        out_type=[
            jax.ShapeDtypeStruct((NC, N, HO), jnp.float32),
            jax.ShapeDtypeStruct((NC, N, 16), jnp.float32),
        ],
        mesh=mesh,
        compiler_params=pltpu.CompilerParams(use_tc_tiling_on_sc=False,
                                             needs_layout_passes=False),
        scratch_types=[
            pltpu.VMEM((CH,), jnp.int32),
            pltpu.VMEM((CH,), jnp.int32),
            pltpu.VMEM((CH, 16), jnp.float32),
            pltpu.VMEM((CH, 16), jnp.float32),
            pltpu.VMEM((CH, HO), jnp.float32),
            pltpu.VMEM((CH, 16), jnp.float32),
            pltpu.VMEM((CH, HO), jnp.float32),
            pltpu.VMEM((16,), jnp.float32),
            pltpu.VMEM_SHARED((NP, HO), jnp.float32),
            pltpu.VMEM_SHARED((NP, 16), jnp.float32),
            pltpu.SemaphoreType.DMA,
        ],
    )
    def k(src_hbm, dst_hbm, t1_hbm, t2_hbm, xw_hbm, c_hbm,
          out_hbm, esum_hbm,
          sidx, didx, t1r, t2r, xwr, exb, msg, cv, out_sp, esum_sp, sem):
        cid = lax.axis_index("c")
        sid = lax.axis_index("s")
        wid = sid * NC + cid

        zero16 = jnp.zeros((16,), jnp.float32)

        def zrow(i, _):
            for j in range(HO // 16):
                msg[i, pl.ds(j * 16, 16)] = zero16
            exb[i, :] = zero16
            return 0

        lax.fori_loop(0, CH, zrow, 0)
        r0 = sid * RPT
        for b in range(RPT // CH):
            pltpu.sync_copy(msg, out_sp.at[pl.ds(r0 + b * CH, CH)])
            pltpu.sync_copy(exb, esum_sp.at[pl.ds(r0 + b * CH, CH)])
        pltpu.sync_copy(c_hbm, cv)
        plsc.subcore_barrier()

        cval = cv[...]
        io = lax.iota(jnp.int32, 16)
        col0 = (io >= 8).astype(jnp.int32)

        def chunk(g, _):
            base = wid * EPW + g * CH
            pltpu.sync_copy(src_hbm.at[pl.ds(base, CH)], sidx)
            pltpu.sync_copy(dst_hbm.at[pl.ds(base, CH)], didx)
            c1 = pltpu.async_copy(t1_hbm.at[sidx], t1r, sem)
            c2 = pltpu.async_copy(t2_hbm.at[didx], t2r, sem)
            c3 = pltpu.async_copy(xw_hbm.at[sidx], xwr, sem)
            c1.wait()
            c2.wait()

            def p1(i, _):
                a = t1r[i, :] + t2r[i, :]
                al = jnp.maximum(a, 0.2 * a)
                exb[i, :] = jnp.exp(al - cval)
                return 0

            lax.fori_loop(0, CH, p1, 0)
            pltpu.sync_copy(exb, esum_sp.at[didx], add=True)
            c3.wait()

            def p2(i, _):
                row = jnp.full((16,), i, jnp.int32)
                for j in range(HO // 16):
                    eb = plsc.load_gather(exb, [row, col0 + 2 * j])
                    msg[i, pl.ds(j * 16, 16)] = xwr[i, pl.ds(j * 16, 16)] * eb
                return 0

            lax.fori_loop(0, CH, p2, 0)
            pltpu.sync_copy(msg, out_sp.at[didx], add=True)
            return 0

        lax.fori_loop(0, NCHUNK, chunk, 0)
        plsc.subcore_barrier()

        nfull = N // RPT
        nlast = N - nfull * RPT

        @pl.when(sid < nfull)
        def _():
            pltpu.sync_copy(out_sp.at[pl.ds(r0, RPT)],
                            out_hbm.at[cid, pl.ds(r0, RPT)])
            pltpu.sync_copy(esum_sp.at[pl.ds(r0, RPT)],
                            esum_hbm.at[cid, pl.ds(r0, RPT)])

        @pl.when(sid == nfull)
        def _():
            pltpu.sync_copy(out_sp.at[pl.ds(nfull * RPT, nlast)],
                            out_hbm.at[cid, pl.ds(nfull * RPT, nlast)])
            pltpu.sync_copy(esum_sp.at[pl.ds(nfull * RPT, nlast)],
                            esum_hbm.at[cid, pl.ds(nfull * RPT, nlast)])

    return k(src, dst, t1, t2, xw, cvec)


def _post_body(outp_ref, esump_ref, xw_ref, t1_ref, t2_ref, c_ref, r8_ref,
               wih_ref, badd_ref, whh_ref, bn_ref, bgat_ref, wlin_ref,
               blin_ref, o_ref, h_ref, gir_scr, giz_scr, gin_scr):
    s = t1_ref[...] + t2_ref[...]
    exs = jnp.exp(jnp.maximum(s, 0.2 * s) - c_ref[...])
    esum = esump_ref[0] + esump_ref[1] + exs
    r8 = r8_ref[...]
    exb = jnp.dot(exs, r8, preferred_element_type=jnp.float32)
    esb = jnp.dot(esum, r8, preferred_element_type=jnp.float32)
    xw = xw_ref[...]
    out = outp_ref[0] + outp_ref[1] + xw * exb
    g = jnp.maximum(out / (esb + 1e-16) + bgat_ref[...], 0.0)
    wih = wih_ref[...]
    badd = badd_ref[...]
    gir_scr[...] = jnp.dot(g, wih[:, 0:16],
                           preferred_element_type=jnp.float32) + badd[:, 0:16]
    giz_scr[...] = jnp.dot(g, wih[:, 16:32],
                           preferred_element_type=jnp.float32) + badd[:, 16:32]
    gin_scr[...] = jnp.dot(g, wih[:, 32:48],
                           preferred_element_type=jnp.float32) + badd[:, 32:48]
    wr = whh_ref[0]
    wz = whh_ref[1]
    wn = whh_ref[2]
    bn = bn_ref[...]

    def sred(p):
        a = p[0:8] + p[8:16]
        b = a[0:4] + a[4:8]
        c = b[0:2] + b[2:4]
        return c[0:1] + c[1:2]

    def step(t, carry):
        hrow, hs = carry
        gr = sred(hs * wr)
        gz = sred(hs * wz)
        gn = sred(hs * wn)
        r = 1.0 / (1.0 + jnp.exp(-(gir_scr[pl.ds(t, 1), :] + gr)))
        z = 1.0 / (1.0 + jnp.exp(-(giz_scr[pl.ds(t, 1), :] + gz)))
        n = jnp.tanh(gin_scr[pl.ds(t, 1), :] + r * (gn + bn))
        hnew = (1.0 - z) * n + z * hrow
        hs_new = jnp.broadcast_to(hnew.reshape(HG, 1), (HG, HG))
        return hnew, hs_new

    h0 = jnp.zeros((1, HG), jnp.float32)
    hs0 = jnp.zeros((HG, HG), jnp.float32)
    h, _ = lax.fori_loop(0, N, step, (h0, hs0))
    h_ref[...] = h
    o_ref[...] = jnp.dot(h, wlin_ref[...],
                         preferred_element_type=jnp.float32) + blin_ref[...]


def _post(outp, esump, xw, t1, t2, cvec, r8, wih, badd, whh, bn, bgat,
          wlin, blin):
    return pl.pallas_call(
        _post_body,
        out_shape=[
            jax.ShapeDtypeStruct((1, 1), jnp.float32),
            jax.ShapeDtypeStruct((1, HG), jnp.float32),
        ],
        scratch_shapes=[pltpu.VMEM((N, HG), jnp.float32),
                        pltpu.VMEM((N, HG), jnp.float32),
                        pltpu.VMEM((N, HG), jnp.float32)],
    )(outp, esump, xw, t1, t2, cvec, r8, wih, badd, whh, bn, bgat, wlin, blin)


def kernel(x, edge_index, W_gat, att_src, att_dst, b_gat,
           W_ih, W_hh, b_ih, b_hh, W_lin, b_lin):
    f32 = jnp.float32
    oh = (jnp.arange(HO)[:, None] // 8 == jnp.arange(16)[None, :]).astype(f32)
    a1 = oh * att_src.reshape(HO, 1)
    a2 = oh * att_dst.reshape(HO, 1)
    r8 = oh.T

    xw, t1, t2, cvec = _pre(x, W_gat, a1, a2)

    pad = E_PAD - E
    src = jnp.concatenate([edge_index[0].astype(jnp.int32),
                           jnp.zeros((pad,), jnp.int32)])
    dst = jnp.concatenate([edge_index[1].astype(jnp.int32),
                           jnp.full((pad,), N, jnp.int32)])
    outp, esump = _edge_sc(src, dst, t1, t2, xw, cvec.reshape(16))

    wih = W_ih.T
    badd = (b_ih + jnp.concatenate([b_hh[0:32], jnp.zeros((16,), f32)])
            ).reshape(1, 48)
    whhT = W_hh.T
    whh = jnp.stack([whhT[:, 0:16], whhT[:, 16:32], whhT[:, 32:48]])
    bn = b_hh[32:48].reshape(1, HG)
    o, h = _post(outp, esump, xw, t1, t2, cvec, r8, wih, badd, whh, bn,
                 b_gat.reshape(1, HO), W_lin.T, b_lin.reshape(1, 1))
    return (o, h.reshape(1, 1, HG))


# speedup vs baseline: 40.5590x; 1.1342x over previous
import functools

import jax
import jax.numpy as jnp
from jax import lax
from jax.experimental import pallas as pl
from jax.experimental.pallas import tpu as pltpu
from jax.experimental.pallas import tpu_sc as plsc

N = 10000
E = 320000
F_IN = 128
HO = 64
HG = 16
NC = 2
NS = 16
NW = NC * NS
CH = 128
NCHUNK = 80
E_PAD = NW * NCHUNK * CH
EPW = E_PAD // NW
NP = 10240
RPT = NP // NS
AW = 80


def _pre_body(x_ref, wg_ref, a1_ref, a2_ref, xw_ref, t1_ref, t2_ref, c_ref):
    xw = jnp.dot(x_ref[...], wg_ref[...], preferred_element_type=jnp.float32)
    xw_ref[...] = xw
    t1 = jnp.dot(xw, a1_ref[...], preferred_element_type=jnp.float32)
    t2 = jnp.dot(xw, a2_ref[...], preferred_element_type=jnp.float32)
    t1_ref[...] = t1
    t2_ref[...] = t2
    m = (jnp.max(t1, axis=0, keepdims=True)
         + jnp.max(t2, axis=0, keepdims=True))
    c_ref[...] = jnp.maximum(m, 0.2 * m)


def _pre(x, wg, a1, a2):
    return pl.pallas_call(
        _pre_body,
        out_shape=[
            jax.ShapeDtypeStruct((N, HO), jnp.float32),
            jax.ShapeDtypeStruct((N, 16), jnp.float32),
            jax.ShapeDtypeStruct((N, 16), jnp.float32),
            jax.ShapeDtypeStruct((1, 16), jnp.float32),
        ],
    )(x, wg, a1, a2)


def _edge_sc(src3, dst3, t1, t2, xw, cvec):
    mesh = plsc.VectorSubcoreMesh(core_axis_name="c", subcore_axis_name="s")

    @functools.partial(
        pl.kernel,
        out_type=[jax.ShapeDtypeStruct((NC, N, AW), jnp.float32)],
        mesh=mesh,
        compiler_params=pltpu.CompilerParams(use_tc_tiling_on_sc=False,
                                             needs_layout_passes=False),
        scratch_types=[
            pltpu.VMEM((NCHUNK + 1, CH), jnp.int32),
            pltpu.VMEM((NCHUNK, CH), jnp.int32),
            pltpu.VMEM((2, CH, 16), jnp.float32),
            pltpu.VMEM((2, CH, 16), jnp.float32),
            pltpu.VMEM((2, CH, HO), jnp.float32),
            pltpu.VMEM((2, CH, AW), jnp.float32),
            pltpu.VMEM((16,), jnp.float32),
            pltpu.VMEM_SHARED((NP, AW), jnp.float32),
            pltpu.SemaphoreType.DMA,
            pltpu.SemaphoreType.DMA,
        ],
    )
    def k(src_hbm, dst_hbm, t1_hbm, t2_hbm, xw_hbm, c_hbm, acc_hbm,
          sidx, didx, t1r, t2r, xwr, comb, cv, acc_sp,
          gsem0, gsem1):
        cid = lax.axis_index("c")
        sid = lax.axis_index("s")
        wid = sid * NC + cid
        gsem = (gsem0, gsem1)

        pltpu.sync_copy(c_hbm, cv)
        pltpu.sync_copy(src_hbm.at[wid], sidx)
        pltpu.sync_copy(dst_hbm.at[wid], didx)

        zero16 = jnp.zeros((16,), jnp.float32)

        @functools.partial(plsc.parallel_loop, 0, CH, unroll=8)
        def _(i):
            for j in range(AW // 16):
                comb[0, i, pl.ds(j * 16, 16)] = zero16

        r0 = sid * RPT
        for b in range(RPT // CH):
            pltpu.sync_copy(comb.at[0], acc_sp.at[pl.ds(r0 + b * CH, CH)])
        plsc.subcore_barrier()

        cval = cv[...]
        io = lax.iota(jnp.int32, 16)
        col0 = (io >= 8).astype(jnp.int32) + HO

        def fire(g, p):
            c1 = pltpu.async_copy(t1_hbm.at[sidx.at[g]], t1r.at[p], gsem[p])
            c2 = pltpu.async_copy(t2_hbm.at[didx.at[g]], t2r.at[p], gsem[p])
            c3 = pltpu.async_copy(xw_hbm.at[sidx.at[g]], xwr.at[p], gsem[p])
            return (c1, c2, c3)

        def compute_and_scatter(g, p):
            @functools.partial(plsc.parallel_loop, 0, CH, unroll=8)
            def _(i):
                a = t1r[p, i, :] + t2r[p, i, :]
                al = jnp.maximum(a, 0.2 * a)
                comb[p, i, pl.ds(HO, 16)] = jnp.exp(al - cval)

            @functools.partial(plsc.parallel_loop, 0, CH, unroll=4)
            def _(i):
                row = jnp.full((16,), i, jnp.int32)
                for j in range(HO // 16):
                    eb = plsc.load_gather(comb.at[p], [row, col0 + 2 * j])
                    comb[p, i, pl.ds(j * 16, 16)] = (
                        xwr[p, i, pl.ds(j * 16, 16)] * eb)

            pltpu.sync_copy(comb.at[p], acc_sp.at[didx.at[g]], add=True)

        def body(g0, _):
            g = 2 * g0
            ca = fire(g, 0)
            cb = fire(g + 1, 1)
            for c in ca:
                c.wait()
            compute_and_scatter(g, 0)
            for c in cb:
                c.wait()
            compute_and_scatter(g + 1, 1)
            return 0

        lax.fori_loop(0, NCHUNK // 2, body, 0)
        plsc.subcore_barrier()

        nfull = N // RPT
        nlast = N - nfull * RPT

        @pl.when(sid < nfull)
        def _():
            pltpu.sync_copy(acc_sp.at[pl.ds(r0, RPT)],
                            acc_hbm.at[cid, pl.ds(r0, RPT)])

        @pl.when(sid == nfull)
        def _():
            pltpu.sync_copy(acc_sp.at[pl.ds(nfull * RPT, nlast)],
                            acc_hbm.at[cid, pl.ds(nfull * RPT, nlast)])

    return k(src3, dst3, t1, t2, xw, cvec)


def _post_body(acc_ref, xw_ref, t1_ref, t2_ref, c_ref, r8_ref,
               wih_ref, badd_ref, whh_ref, bn_ref, bgat_ref, wlin_ref,
               blin_ref, o_ref, h_ref, gir_scr, giz_scr, gin_scr):
    s = t1_ref[...] + t2_ref[...]
    exs = jnp.exp(jnp.maximum(s, 0.2 * s) - c_ref[...])
    acc = acc_ref[0] + acc_ref[1]
    esum = acc[:, HO:AW] + exs
    r8 = r8_ref[...]
    exb = jnp.dot(exs, r8, preferred_element_type=jnp.float32)
    esb = jnp.dot(esum, r8, preferred_element_type=jnp.float32)
    xw = xw_ref[...]
    out = acc[:, 0:HO] + xw * exb
    g = jnp.maximum(out / (esb + 1e-16) + bgat_ref[...], 0.0)
    wih = wih_ref[...]
    badd = badd_ref[...]
    gir_scr[...] = jnp.dot(g, wih[:, 0:16],
                           preferred_element_type=jnp.float32) + badd[:, 0:16]
    giz_scr[...] = jnp.dot(g, wih[:, 16:32],
                           preferred_element_type=jnp.float32) + badd[:, 16:32]
    gin_scr[...] = jnp.dot(g, wih[:, 32:48],
                           preferred_element_type=jnp.float32) + badd[:, 32:48]
    wr = whh_ref[0]
    wz = whh_ref[1]
    wn = whh_ref[2]
    bn = bn_ref[...]

    def sred(p):
        a = p[0:8] + p[8:16]
        b = a[0:4] + a[4:8]
        c = b[0:2] + b[2:4]
        return c[0:1] + c[1:2]

    def step(t, carry):
        hrow, hs = carry
        gr = sred(hs * wr)
        gz = sred(hs * wz)
        gn = sred(hs * wn)
        r = 1.0 / (1.0 + jnp.exp(-(gir_scr[pl.ds(t, 1), :] + gr)))
        z = 1.0 / (1.0 + jnp.exp(-(giz_scr[pl.ds(t, 1), :] + gz)))
        n = jnp.tanh(gin_scr[pl.ds(t, 1), :] + r * (gn + bn))
        hnew = (1.0 - z) * n + z * hrow
        hs_new = jnp.broadcast_to(hnew.reshape(HG, 1), (HG, HG))
        return hnew, hs_new

    h0 = jnp.zeros((1, HG), jnp.float32)
    hs0 = jnp.zeros((HG, HG), jnp.float32)
    h, _ = lax.fori_loop(0, N, step, (h0, hs0))
    h_ref[...] = h
    o_ref[...] = jnp.dot(h, wlin_ref[...],
                         preferred_element_type=jnp.float32) + blin_ref[...]


def _post(acc, xw, t1, t2, cvec, r8, wih, badd, whh, bn, bgat,
          wlin, blin):
    return pl.pallas_call(
        _post_body,
        out_shape=[
            jax.ShapeDtypeStruct((1, 1), jnp.float32),
            jax.ShapeDtypeStruct((1, HG), jnp.float32),
        ],
        scratch_shapes=[pltpu.VMEM((N, HG), jnp.float32),
                        pltpu.VMEM((N, HG), jnp.float32),
                        pltpu.VMEM((N, HG), jnp.float32)],
    )(acc, xw, t1, t2, cvec, r8, wih, badd, whh, bn, bgat, wlin, blin)


def kernel(x, edge_index, W_gat, att_src, att_dst, b_gat,
           W_ih, W_hh, b_ih, b_hh, W_lin, b_lin):
    f32 = jnp.float32
    oh = (jnp.arange(HO)[:, None] // 8 == jnp.arange(16)[None, :]).astype(f32)
    a1 = oh * att_src.reshape(HO, 1)
    a2 = oh * att_dst.reshape(HO, 1)
    r8 = oh.T

    xw, t1, t2, cvec = _pre(x, W_gat, a1, a2)

    pad = E_PAD - E
    src = jnp.concatenate([edge_index[0].astype(jnp.int32),
                           jnp.zeros((pad,), jnp.int32)])
    dst = jnp.concatenate([edge_index[1].astype(jnp.int32),
                           jnp.full((pad,), N, jnp.int32)])
    src3 = jnp.concatenate([src.reshape(NW, NCHUNK, CH),
                            jnp.zeros((NW, 1, CH), jnp.int32)], axis=1)
    dst3 = dst.reshape(NW, NCHUNK, CH)
    t2p = jnp.concatenate([t2, jnp.zeros((16, 16), f32)], axis=0)
    (acc,) = _edge_sc(src3, dst3, t1, t2p, xw, cvec.reshape(16))

    wih = W_ih.T
    badd = (b_ih + jnp.concatenate([b_hh[0:32], jnp.zeros((16,), f32)])
            ).reshape(1, 48)
    whhT = W_hh.T
    whh = jnp.stack([whhT[:, 0:16], whhT[:, 16:32], whhT[:, 32:48]])
    bn = b_hh[32:48].reshape(1, HG)
    o, h = _post(acc, xw, t1, t2, cvec, r8, wih, badd, whh, bn,
                 b_gat.reshape(1, HO), W_lin.T, b_lin.reshape(1, 1))
    return (o, h.reshape(1, 1, HG))


# speedup vs baseline: 42.2739x; 1.0423x over previous
import functools

import jax
import jax.numpy as jnp
from jax import lax
from jax.experimental import pallas as pl
from jax.experimental.pallas import tpu as pltpu
from jax.experimental.pallas import tpu_sc as plsc

N = 10000
E = 320000
F_IN = 128
HO = 64
HG = 16
NC = 2
NS = 16
NW = NC * NS
CH = 128
NCHUNK = 80
E_PAD = NW * NCHUNK * CH
EPW = E_PAD // NW
NP = 10240
RPT = NP // NS
AW = 80


def _pre_body(x_ref, wg_ref, a1_ref, a2_ref, xw_ref, t1_ref, t2_ref, c_ref):
    xw = jnp.dot(x_ref[...], wg_ref[...], preferred_element_type=jnp.float32)
    xw_ref[...] = xw
    t1 = jnp.dot(xw, a1_ref[...], preferred_element_type=jnp.float32)
    t2 = jnp.dot(xw, a2_ref[...], preferred_element_type=jnp.float32)
    t1_ref[...] = t1
    t2_ref[...] = t2
    m = (jnp.max(t1, axis=0, keepdims=True)
         + jnp.max(t2, axis=0, keepdims=True))
    c_ref[...] = jnp.maximum(m, 0.2 * m)


def _pre(x, wg, a1, a2):
    return pl.pallas_call(
        _pre_body,
        out_shape=[
            jax.ShapeDtypeStruct((N, HO), jnp.float32),
            jax.ShapeDtypeStruct((N, 16), jnp.float32),
            jax.ShapeDtypeStruct((N, 16), jnp.float32),
            jax.ShapeDtypeStruct((1, 16), jnp.float32),
        ],
    )(x, wg, a1, a2)


def _edge_sc(src3, dst3, t1, t2, xw, cvec):
    mesh = plsc.VectorSubcoreMesh(core_axis_name="c", subcore_axis_name="s")

    @functools.partial(
        pl.kernel,
        out_type=[jax.ShapeDtypeStruct((NC, N, AW), jnp.float32)],
        mesh=mesh,
        compiler_params=pltpu.CompilerParams(use_tc_tiling_on_sc=False,
                                             needs_layout_passes=False),
        scratch_types=[
            pltpu.VMEM((NCHUNK + 1, CH), jnp.int32),
            pltpu.VMEM((NCHUNK, CH), jnp.int32),
            pltpu.VMEM((2, CH, 16), jnp.float32),
            pltpu.VMEM((2, CH, 16), jnp.float32),
            pltpu.VMEM((2, CH, HO), jnp.float32),
            pltpu.VMEM((2, CH, AW), jnp.float32),
            pltpu.VMEM((16,), jnp.float32),
            pltpu.VMEM_SHARED((NP, AW), jnp.float32),
            pltpu.SemaphoreType.DMA,
            pltpu.SemaphoreType.DMA,
        ],
    )
    def k(src_hbm, dst_hbm, t1_hbm, t2_hbm, xw_hbm, c_hbm, acc_hbm,
          sidx, didx, t1r, t2r, xwr, comb, cv, acc_sp,
          gsem0, gsem1):
        cid = lax.axis_index("c")
        sid = lax.axis_index("s")
        wid = sid * NC + cid
        gsem = (gsem0, gsem1)

        pltpu.sync_copy(c_hbm, cv)
        pltpu.sync_copy(src_hbm.at[wid], sidx)
        pltpu.sync_copy(dst_hbm.at[wid], didx)

        zero16 = jnp.zeros((16,), jnp.float32)

        @functools.partial(plsc.parallel_loop, 0, CH, unroll=8)
        def _(i):
            for j in range(AW // 16):
                comb[0, i, pl.ds(j * 16, 16)] = zero16

        r0 = sid * RPT
        for b in range(RPT // CH):
            pltpu.sync_copy(comb.at[0], acc_sp.at[pl.ds(r0 + b * CH, CH)])
        plsc.subcore_barrier()

        cval = cv[...]
        io = lax.iota(jnp.int32, 16)
        col0 = (io >= 8).astype(jnp.int32) + HO

        def fire(g, p):
            c1 = pltpu.async_copy(t1_hbm.at[sidx.at[g]], t1r.at[p], gsem[p])
            c2 = pltpu.async_copy(t2_hbm.at[didx.at[g]], t2r.at[p], gsem[p])
            c3 = pltpu.async_copy(xw_hbm.at[sidx.at[g]], xwr.at[p], gsem[p])
            return (c1, c2, c3)

        def compute_and_scatter(g, p):
            @functools.partial(plsc.parallel_loop, 0, CH, unroll=8)
            def _(i):
                a = t1r[p, i, :] + t2r[p, i, :]
                al = jnp.maximum(a, 0.2 * a)
                comb[p, i, pl.ds(HO, 16)] = jnp.exp(al - cval)

            @functools.partial(plsc.parallel_loop, 0, CH, unroll=4)
            def _(i):
                row = jnp.full((16,), i, jnp.int32)
                for j in range(HO // 16):
                    eb = plsc.load_gather(comb.at[p], [row, col0 + 2 * j])
                    comb[p, i, pl.ds(j * 16, 16)] = (
                        xwr[p, i, pl.ds(j * 16, 16)] * eb)

            pltpu.sync_copy(comb.at[p], acc_sp.at[didx.at[g]], add=True)

        def body(g0, _):
            g = 2 * g0
            ca = fire(g, 0)
            cb = fire(g + 1, 1)
            for c in ca:
                c.wait()
            compute_and_scatter(g, 0)
            for c in cb:
                c.wait()
            compute_and_scatter(g + 1, 1)
            return 0

        lax.fori_loop(0, NCHUNK // 2, body, 0)
        plsc.subcore_barrier()

        nfull = N // RPT
        nlast = N - nfull * RPT

        @pl.when(sid < nfull)
        def _():
            pltpu.sync_copy(acc_sp.at[pl.ds(r0, RPT)],
                            acc_hbm.at[cid, pl.ds(r0, RPT)])

        @pl.when(sid == nfull)
        def _():
            pltpu.sync_copy(acc_sp.at[pl.ds(nfull * RPT, nlast)],
                            acc_hbm.at[cid, pl.ds(nfull * RPT, nlast)])

    return k(src3, dst3, t1, t2, xw, cvec)


def _post_body(acc_ref, xw_ref, t1_ref, t2_ref, c_ref, r8_ref,
               wih_ref, badd_ref, whh_ref, bn_ref, bgat_ref, wlin_ref,
               blin_ref, o_ref, h_ref, gir_scr, giz_scr, gin_scr):
    s = t1_ref[...] + t2_ref[...]
    exs = jnp.exp(jnp.maximum(s, 0.2 * s) - c_ref[...])
    acc = acc_ref[0] + acc_ref[1]
    esum = acc[:, HO:AW] + exs
    r8 = r8_ref[...]
    exb = jnp.dot(exs, r8, preferred_element_type=jnp.float32)
    esb = jnp.dot(esum, r8, preferred_element_type=jnp.float32)
    xw = xw_ref[...]
    out = acc[:, 0:HO] + xw * exb
    g = jnp.maximum(out / (esb + 1e-16) + bgat_ref[...], 0.0)
    wih = wih_ref[...]
    badd = badd_ref[...]
    gir_scr[pl.ds(0, N), :] = jnp.dot(
        g, wih[:, 0:16], preferred_element_type=jnp.float32) + badd[:, 0:16]
    giz_scr[pl.ds(0, N), :] = jnp.dot(
        g, wih[:, 16:32], preferred_element_type=jnp.float32) + badd[:, 16:32]
    gin_scr[pl.ds(0, N), :] = jnp.dot(
        g, wih[:, 32:48], preferred_element_type=jnp.float32) + badd[:, 32:48]
    wr = whh_ref[0]
    wz = whh_ref[1]
    wn = whh_ref[2]
    bn = bn_ref[...]

    def sred(p):
        a = p[0:8] + p[8:16]
        b = a[0:4] + a[4:8]
        c = b[0:2] + b[2:4]
        return c[0:1] + c[1:2]

    def load_gi(t):
        return (gir_scr[pl.ds(t, 1), :], giz_scr[pl.ds(t, 1), :],
                gin_scr[pl.ds(t, 1), :])

    def one(t, hrow, hs, gi):
        gir, giz, gin = gi
        gi_next = load_gi(t + 1)
        gr = sred(hs * wr)
        gz = sred(hs * wz)
        gn = sred(hs * wn)
        r = 0.5 + 0.5 * jnp.tanh(0.5 * (gir + gr))
        z = 0.5 + 0.5 * jnp.tanh(0.5 * (giz + gz))
        n = jnp.tanh(gin + r * (gn + bn))
        hnew = (1.0 - z) * n + z * hrow
        hs_new = jnp.broadcast_to(hnew.reshape(HG, 1), (HG, HG))
        return hnew, hs_new, gi_next

    def step(t2, carry):
        hrow, hs, gi = carry
        hrow, hs, gi = one(2 * t2, hrow, hs, gi)
        hrow, hs, gi = one(2 * t2 + 1, hrow, hs, gi)
        return hrow, hs, gi

    h0 = jnp.zeros((1, HG), jnp.float32)
    hs0 = jnp.zeros((HG, HG), jnp.float32)
    h, _, _ = lax.fori_loop(0, N // 2, step, (h0, hs0, load_gi(0)))
    h_ref[...] = h
    o_ref[...] = jnp.dot(h, wlin_ref[...],
                         preferred_element_type=jnp.float32) + blin_ref[...]


def _post(acc, xw, t1, t2, cvec, r8, wih, badd, whh, bn, bgat,
          wlin, blin):
    return pl.pallas_call(
        _post_body,
        out_shape=[
            jax.ShapeDtypeStruct((1, 1), jnp.float32),
            jax.ShapeDtypeStruct((1, HG), jnp.float32),
        ],
        scratch_shapes=[pltpu.VMEM((N + 8, HG), jnp.float32),
                        pltpu.VMEM((N + 8, HG), jnp.float32),
                        pltpu.VMEM((N + 8, HG), jnp.float32)],
    )(acc, xw, t1, t2, cvec, r8, wih, badd, whh, bn, bgat, wlin, blin)


def kernel(x, edge_index, W_gat, att_src, att_dst, b_gat,
           W_ih, W_hh, b_ih, b_hh, W_lin, b_lin):
    f32 = jnp.float32
    oh = (jnp.arange(HO)[:, None] // 8 == jnp.arange(16)[None, :]).astype(f32)
    a1 = oh * att_src.reshape(HO, 1)
    a2 = oh * att_dst.reshape(HO, 1)
    r8 = oh.T

    xw, t1, t2, cvec = _pre(x, W_gat, a1, a2)

    pad = E_PAD - E
    src = jnp.concatenate([edge_index[0].astype(jnp.int32),
                           jnp.zeros((pad,), jnp.int32)])
    dst = jnp.concatenate([edge_index[1].astype(jnp.int32),
                           jnp.full((pad,), N, jnp.int32)])
    src3 = jnp.concatenate([src.reshape(NW, NCHUNK, CH),
                            jnp.zeros((NW, 1, CH), jnp.int32)], axis=1)
    dst3 = dst.reshape(NW, NCHUNK, CH)
    t2p = jnp.concatenate([t2, jnp.zeros((16, 16), f32)], axis=0)
    (acc,) = _edge_sc(src3, dst3, t1, t2p, xw, cvec.reshape(16))

    wih = W_ih.T
    badd = (b_ih + jnp.concatenate([b_hh[0:32], jnp.zeros((16,), f32)])
            ).reshape(1, 48)
    whhT = W_hh.T
    whh = jnp.stack([whhT[:, 0:16], whhT[:, 16:32], whhT[:, 32:48]])
    bn = b_hh[32:48].reshape(1, HG)
    o, h = _post(acc, xw, t1, t2, cvec, r8, wih, badd, whh, bn,
                 b_gat.reshape(1, HO), W_lin.T, b_lin.reshape(1, 1))
    return (o, h.reshape(1, 1, HG))
